# Initial kernel scaffold; baseline (speedup 1.0000x reference)
import jax
import jax.numpy as jnp
from jax.experimental import pallas as pl


def kernel(H, batch_idx):
    raise NotImplementedError("write your pallas kernel here")


# speedup vs baseline: 4.1920x; 4.1920x over previous
import functools

import jax
import jax.numpy as jnp
from jax import lax
from jax.experimental import pallas as pl
from jax.experimental.pallas import tpu as pltpu
from jax.experimental.pallas import tpu_sc as plsc

NSEG = 1024
D = 128
NROWS = 100000
CHUNK = 128
NFULL = NROWS // CHUNK
TAIL = NROWS - NFULL * CHUNK
NCORES = 2
NSUB = 16
NW = NCORES * NSUB
ROUNDS = -(-(NFULL + 1) // NW)
SEG_PER_SUB = NSEG // NSUB


def _sc_partials(H, idx, zeros_stripe):
    mesh = plsc.VectorSubcoreMesh(core_axis_name="c", subcore_axis_name="s")

    @functools.partial(
        pl.kernel,
        mesh=mesh,
        out_type=jax.ShapeDtypeStruct((NCORES, NSEG, D), jnp.float32),
        scratch_types=[
            pltpu.VMEM((CHUNK, D), jnp.float32),
            pltpu.VMEM((CHUNK,), jnp.int32),
            pltpu.VMEM((TAIL, D), jnp.float32),
            pltpu.VMEM((TAIL,), jnp.int32),
            pltpu.VMEM_SHARED((NSEG, D), jnp.float32),
        ],
    )
    def k(h_hbm, idx_hbm, z_hbm, out_hbm, rows_v, idx_v, rows_t, idx_t, acc_sh):
        c = lax.axis_index("c")
        s = lax.axis_index("s")
        t = s * NCORES + c

        pltpu.sync_copy(z_hbm, rows_v.at[pl.ds(0, SEG_PER_SUB)])
        pltpu.sync_copy(rows_v.at[pl.ds(0, SEG_PER_SUB)],
                        acc_sh.at[pl.ds(s * SEG_PER_SUB, SEG_PER_SUB)])
        plsc.subcore_barrier()

        def body(r, carry):
            cid = t + r * NW

            @pl.when(cid < NFULL)
            def _():
                base = cid * CHUNK
                pltpu.sync_copy(h_hbm.at[pl.ds(base, CHUNK)], rows_v)
                pltpu.sync_copy(idx_hbm.at[pl.ds(base, CHUNK)], idx_v)
                pltpu.sync_copy(rows_v, acc_sh.at[idx_v], add=True)

            @pl.when(cid == NFULL)
            def _():
                base = NFULL * CHUNK
                pltpu.sync_copy(h_hbm.at[pl.ds(base, TAIL)], rows_t)
                pltpu.sync_copy(idx_hbm.at[pl.ds(base, TAIL)], idx_t)
                pltpu.sync_copy(rows_t, acc_sh.at[idx_t], add=True)

            return carry

        lax.fori_loop(0, ROUNDS, body, 0)
        plsc.subcore_barrier()

        pltpu.sync_copy(acc_sh.at[pl.ds(s * SEG_PER_SUB, SEG_PER_SUB)],
                        rows_v.at[pl.ds(0, SEG_PER_SUB)])
        pltpu.sync_copy(rows_v.at[pl.ds(0, SEG_PER_SUB)],
                        out_hbm.at[c, pl.ds(s * SEG_PER_SUB, SEG_PER_SUB)])

    return k(H, idx, zeros_stripe)


def _combine(partials):
    def body(p_ref, o_ref):
        o_ref[...] = p_ref[0] + p_ref[1]

    return pl.pallas_call(
        body,
        out_shape=jax.ShapeDtypeStruct((NSEG, D), jnp.float32),
    )(partials)


def kernel(H, batch_idx):
    idx = batch_idx.astype(jnp.int32)
    zeros_stripe = jnp.zeros((SEG_PER_SUB, D), jnp.float32)
    partials = _sc_partials(H, idx, zeros_stripe)
    return _combine(partials)


# speedup vs baseline: 6.4039x; 1.5276x over previous
import functools

import jax
import jax.numpy as jnp
from jax import lax
from jax.experimental import pallas as pl
from jax.experimental.pallas import tpu as pltpu
from jax.experimental.pallas import tpu_sc as plsc

NSEG = 1024
D = 128
NROWS = 100000
CHUNK = 128
NFULL = NROWS // CHUNK
TAIL = NROWS - NFULL * CHUNK
NCORES = 2
NSUB = 16
NW = NCORES * NSUB
BASE_SPAN = NFULL // NW
EXTRA = NFULL - BASE_SPAN * NW
MAX_SPAN = BASE_SPAN + 1
SEG_PER_SUB = NSEG // NSUB


def _sc_partials(H, idx1d, idx_tail, zeros_stripe):
    mesh = plsc.VectorSubcoreMesh(core_axis_name="c", subcore_axis_name="s")

    @functools.partial(
        pl.kernel,
        mesh=mesh,
        out_type=jax.ShapeDtypeStruct((NCORES, NSEG, D), jnp.float32),
        scratch_types=[
            pltpu.VMEM((2, CHUNK, D), jnp.float32),
            pltpu.VMEM((2, CHUNK), jnp.int32),
            pltpu.VMEM((TAIL, D), jnp.float32),
            pltpu.VMEM((TAIL,), jnp.int32),
            pltpu.VMEM_SHARED((NSEG, D), jnp.float32),
            pltpu.SemaphoreType.DMA((2,)),
        ],
    )
    def k(h_hbm, idx_hbm, idxt_hbm, z_hbm, out_hbm,
          rows_v, idx_v, rows_t, idx_t, acc_sh, sem):
        c = lax.axis_index("c")
        s = lax.axis_index("s")
        t = s * NCORES + c

        pltpu.sync_copy(z_hbm, rows_v.at[0, pl.ds(0, SEG_PER_SUB)])
        pltpu.sync_copy(rows_v.at[0, pl.ds(0, SEG_PER_SUB)],
                        acc_sh.at[pl.ds(s * SEG_PER_SUB, SEG_PER_SUB)])

        start = t * BASE_SPAN + jnp.minimum(t, EXTRA)
        n_t = jnp.where(t < EXTRA, MAX_SPAN, BASE_SPAN)
        plsc.subcore_barrier()

        def issue(j, b):
            pltpu.async_copy(
                h_hbm.at[pl.ds((start + j) * CHUNK, CHUNK)],
                rows_v.at[b], sem.at[b])
            pltpu.async_copy(
                idx_hbm.at[pl.ds((start + j) * CHUNK, CHUNK)],
                idx_v.at[b], sem.at[b])

        issue(0, 0)

        def body(j, carry):
            b = lax.rem(j, 2)

            @pl.when(j + 1 < n_t)
            def _():
                issue(j + 1, lax.rem(j + 1, 2))

            pltpu.make_async_copy(
                h_hbm.at[pl.ds(0, CHUNK)], rows_v.at[b], sem.at[b]).wait()
            pltpu.make_async_copy(
                idx_hbm.at[pl.ds(0, CHUNK)], idx_v.at[b], sem.at[b]).wait()
            pltpu.sync_copy(rows_v.at[b], acc_sh.at[idx_v.at[b]], add=True)
            return carry

        lax.fori_loop(0, n_t, body, 0)

        @pl.when(t == NW - 1)
        def _():
            base = NFULL * CHUNK
            pltpu.sync_copy(h_hbm.at[pl.ds(base, TAIL)], rows_t)
            pltpu.sync_copy(idxt_hbm, idx_t)
            pltpu.sync_copy(rows_t, acc_sh.at[idx_t], add=True)

        plsc.subcore_barrier()

        pltpu.sync_copy(acc_sh.at[pl.ds(s * SEG_PER_SUB, SEG_PER_SUB)],
                        rows_v.at[0, pl.ds(0, SEG_PER_SUB)])
        pltpu.sync_copy(rows_v.at[0, pl.ds(0, SEG_PER_SUB)],
                        out_hbm.at[c, pl.ds(s * SEG_PER_SUB, SEG_PER_SUB)])

    return k(H, idx1d, idx_tail, zeros_stripe)


def _combine(partials):
    def body(p_ref, o_ref):
        o_ref[...] = p_ref[0] + p_ref[1]

    return pl.pallas_call(
        body,
        out_shape=jax.ShapeDtypeStruct((NSEG, D), jnp.float32),
    )(partials)


def kernel(H, batch_idx):
    idx = batch_idx.astype(jnp.int32)
    idx_tail = idx[NFULL * CHUNK:]
    zeros_stripe = jnp.zeros((SEG_PER_SUB, D), jnp.float32)
    partials = _sc_partials(H, idx, idx_tail, zeros_stripe)
    return _combine(partials)
